# baseline (speedup 1.0000x reference)
import functools

import jax
import jax.numpy as jnp
from jax import lax
from jax.experimental import layout as jlayout
from jax.experimental import pallas as pl
from jax.experimental.pallas import tpu as pltpu
from jax.experimental.pallas import tpu_sc as plsc

BATCH = 16384
HIST = 50
DIM = 32
NC = 2
NS = 16
NW = NC * NS
R = 8
NG = BATCH // (NW * R)
NI = NG // 2


def _emb_kernel(table_hbm, x_hbm, out_hbm, idx_v, rows_v, gsem0, gsem1):
    wid = lax.axis_index("s") * NC + lax.axis_index("c")
    rbase = wid * NG * R

    def fire(r0, b, gsem):
        pltpu.sync_copy(x_hbm.at[pl.ds(r0, R)], idx_v.at[b])
        return [
            pltpu.async_copy(
                table_hbm.at[idx_v.at[b, r]], rows_v.at[b, r], gsem
            )
            for r in range(R)
        ]

    def drain(r0, b, descs):
        for d in descs:
            d.wait()
        pltpu.sync_copy(rows_v.at[b], out_hbm.at[pl.ds(r0, R)])

    def iter_body(i, carry):
        r0 = rbase + 2 * i * R
        d0 = fire(r0, 0, gsem0)
        d1 = fire(r0 + R, 1, gsem1)
        drain(r0, 0, d0)
        drain(r0 + R, 1, d1)
        return carry

    lax.fori_loop(0, NI, iter_body, 0)


@jax.jit
def kernel(x, table):
    assert x.shape == (BATCH, HIST) and table.shape[1] == DIM
    run = functools.partial(
        pl.kernel,
        out_type=jax.ShapeDtypeStruct((BATCH, HIST, DIM), jnp.float32),
        mesh=plsc.VectorSubcoreMesh(core_axis_name="c", subcore_axis_name="s"),
        compiler_params=pltpu.CompilerParams(use_tc_tiling_on_sc=False),
        scratch_types=[
            pltpu.VMEM((2, R, HIST), jnp.int32),
            pltpu.VMEM((2, R, HIST, DIM), jnp.float32),
            pltpu.SemaphoreType.DMA,
            pltpu.SemaphoreType.DMA,
        ],
    )(_emb_kernel)
    out = run(table, x.astype(jnp.int32))
    one = lax.optimization_barrier(jnp.float32(1.0))
    return out * one


# speedup vs baseline: 1.3108x; 1.3108x over previous
import functools

import jax
import jax.numpy as jnp
from jax import lax
from jax.experimental import layout as jlayout
from jax.experimental import pallas as pl
from jax.experimental.pallas import tpu as pltpu
from jax.experimental.pallas import tpu_sc as plsc

BATCH = 16384
HIST = 50
DIM = 32
NC = 2
NS = 16
NW = NC * NS
R = 16
NG = BATCH // (NW * R)
NI = NG // 2


def _emb_kernel(table_hbm, x_hbm, out_hbm, idx_v, rows_v, gsem0, gsem1):
    wid = lax.axis_index("s") * NC + lax.axis_index("c")
    rbase = wid * NG * R

    def fire(r0, b, gsem):
        pltpu.sync_copy(x_hbm.at[pl.ds(r0, R)], idx_v.at[b])
        return [
            pltpu.async_copy(
                table_hbm.at[idx_v.at[b, r]], rows_v.at[b, r], gsem
            )
            for r in range(R)
        ]

    def drain(r0, b, descs):
        for d in descs:
            d.wait()
        pltpu.sync_copy(rows_v.at[b], out_hbm.at[pl.ds(r0, R)])

    def iter_body(i, carry):
        r0 = rbase + 2 * i * R
        d0 = fire(r0, 0, gsem0)
        d1 = fire(r0 + R, 1, gsem1)
        drain(r0, 0, d0)
        drain(r0 + R, 1, d1)
        return carry

    lax.fori_loop(0, NI, iter_body, 0)


@jax.jit
def kernel(x, table):
    assert x.shape == (BATCH, HIST) and table.shape[1] == DIM
    run = functools.partial(
        pl.kernel,
        out_type=jax.ShapeDtypeStruct((BATCH, HIST, DIM), jnp.float32),
        mesh=plsc.VectorSubcoreMesh(core_axis_name="c", subcore_axis_name="s"),
        compiler_params=pltpu.CompilerParams(use_tc_tiling_on_sc=False),
        scratch_types=[
            pltpu.VMEM((2, R, HIST), jnp.int32),
            pltpu.VMEM((2, R, HIST, DIM), jnp.float32),
            pltpu.SemaphoreType.DMA,
            pltpu.SemaphoreType.DMA,
        ],
    )(_emb_kernel)
    return run(table, x.astype(jnp.int32))


# speedup vs baseline: 1.3183x; 1.0057x over previous
import functools

import jax
import jax.numpy as jnp
from jax import lax
from jax.experimental import layout as jlayout
from jax.experimental import pallas as pl
from jax.experimental.pallas import tpu as pltpu
from jax.experimental.pallas import tpu_sc as plsc

BATCH = 16384
HIST = 50
DIM = 32
NC = 2
NS = 16
NW = NC * NS
R = 32
NG = BATCH // (NW * R)
NI = NG // 2


def _emb_kernel(table_hbm, x_hbm, out_hbm, idx_v, rows_v, gsem0, gsem1):
    wid = lax.axis_index("s") * NC + lax.axis_index("c")
    rbase = wid * NG * R

    def fire(r0, b, gsem):
        pltpu.sync_copy(x_hbm.at[pl.ds(r0, R)], idx_v.at[b])
        return [
            pltpu.async_copy(
                table_hbm.at[idx_v.at[b, r]], rows_v.at[b, r], gsem
            )
            for r in range(R)
        ]

    def drain(r0, b, descs):
        for d in descs:
            d.wait()
        pltpu.sync_copy(rows_v.at[b], out_hbm.at[pl.ds(r0, R)])

    def iter_body(i, carry):
        r0 = rbase + 2 * i * R
        d0 = fire(r0, 0, gsem0)
        d1 = fire(r0 + R, 1, gsem1)
        drain(r0, 0, d0)
        drain(r0 + R, 1, d1)
        return carry

    lax.fori_loop(0, NI, iter_body, 0)


@jax.jit
def kernel(x, table):
    assert x.shape == (BATCH, HIST) and table.shape[1] == DIM
    run = functools.partial(
        pl.kernel,
        out_type=jax.ShapeDtypeStruct((BATCH, HIST, DIM), jnp.float32),
        mesh=plsc.VectorSubcoreMesh(core_axis_name="c", subcore_axis_name="s"),
        compiler_params=pltpu.CompilerParams(use_tc_tiling_on_sc=False),
        scratch_types=[
            pltpu.VMEM((2, R, HIST), jnp.int32),
            pltpu.VMEM((2, R, HIST, DIM), jnp.float32),
            pltpu.SemaphoreType.DMA,
            pltpu.SemaphoreType.DMA,
        ],
    )(_emb_kernel)
    return run(table, x.astype(jnp.int32))
